# Initial kernel scaffold; baseline (speedup 1.0000x reference)
import jax
import jax.numpy as jnp
from jax.experimental import pallas as pl


def kernel(feat, label, memory, source_memo):
    raise NotImplementedError("write your pallas kernel here")


# speedup vs baseline: 2.8675x; 2.8675x over previous
import functools

import jax
import jax.numpy as jnp
from jax import lax
from jax.experimental import pallas as pl
from jax.experimental.pallas import tpu as pltpu

B = 16384
D = 1024
C = 1000
CP = 1024
M = 2 * CP
RB = 1024
NBLK = B // RB
NPAD = 2 * (CP - C)


def _stats_body(feat_ref, lbl_ref, mem_ref, src_ref,
                featn_ref, memot_ref, sums_ref, counts_ref):
    i = pl.program_id(0)

    x = feat_ref[...]
    ss = jnp.sum(x * x, axis=1, keepdims=True)
    inv = 1.0 / jnp.maximum(jnp.sqrt(ss), 1e-12)
    xn = x * inv
    xnb = xn.astype(jnp.bfloat16)
    featn_ref[...] = xnb

    lbl = lbl_ref[0, 0, :]
    cls = lax.broadcasted_iota(jnp.int32, (CP, RB), 0)
    oh = cls == lbl[None, :]

    @pl.when(i == 0)
    def _():
        sums_ref[...] = jnp.zeros_like(sums_ref)
        counts_ref[...] = jnp.zeros_like(counts_ref)

    sums_ref[...] += lax.dot_general(
        oh.astype(jnp.bfloat16), xnb,
        (((1,), (0,)), ((), ())), preferred_element_type=jnp.float32)
    counts_ref[...] += jnp.sum(oh.astype(jnp.float32), axis=1, keepdims=True)

    @pl.when(i == NBLK - 1)
    def _():
        sums = sums_ref[...]
        counts = counts_ref[...]
        present = counts > 0.0
        snorm = jnp.sqrt(jnp.sum(sums * sums, axis=1, keepdims=True))
        bc = jnp.where(present, sums / jnp.maximum(snorm, 1e-12), 0.0)
        mem = mem_ref[...]
        uw = jnp.sum(mem * bc, axis=1, keepdims=True)
        uw = 1.0 - (1.0 - uw) * present.astype(jnp.float32)
        nm = uw * mem + (1.0 - uw) * bc
        nnorm = jnp.sqrt(jnp.sum(nm * nm, axis=1, keepdims=True))
        nm = nm / jnp.maximum(nnorm, 1e-12)
        memot_ref[:, 0:CP] = jnp.transpose(nm).astype(jnp.bfloat16)
        memot_ref[:, CP:M] = jnp.transpose(src_ref[...]).astype(jnp.bfloat16)


def _loss_body(featn_ref, memot_ref, lbl_ref, out_ref, acc_ref):
    i = pl.program_id(0)
    x = featn_ref[...]
    logits = lax.dot_general(
        x, memot_ref[...],
        (((1,), (0,)), ((), ())), preferred_element_type=jnp.float32)
    es = jnp.sum(jnp.exp(logits), axis=1, keepdims=True) - float(NPAD)
    lse = jnp.log(es)
    lbl = lbl_ref[0, 0, :]
    col = lax.broadcasted_iota(jnp.int32, (RB, CP), 1)
    ll = jnp.sum(jnp.where(col == lbl[:, None], logits[:, 0:CP], 0.0),
                 axis=1, keepdims=True)

    @pl.when(i == 0)
    def _():
        acc_ref[...] = jnp.zeros_like(acc_ref)

    acc_ref[...] += (lse - ll).reshape(8, RB // 8)

    @pl.when(i == NBLK - 1)
    def _():
        out_ref[...] = (jnp.sum(acc_ref[...]) / float(B)).reshape(1, 1)


@jax.jit
def kernel(feat, label, memory, source_memo):
    lbl3 = label.astype(jnp.int32).reshape(NBLK, 1, RB)
    mem_p = jnp.pad(memory, ((0, CP - C), (0, 0)))
    src_p = jnp.pad(source_memo, ((0, CP - C), (0, 0)))

    featn, memot = pl.pallas_call(
        _stats_body,
        grid=(NBLK,),
        in_specs=[
            pl.BlockSpec((RB, D), lambda i: (i, 0)),
            pl.BlockSpec((1, 1, RB), lambda i: (i, 0, 0)),
            pl.BlockSpec((CP, D), lambda i: (0, 0)),
            pl.BlockSpec((CP, D), lambda i: (0, 0)),
        ],
        out_specs=[
            pl.BlockSpec((RB, D), lambda i: (i, 0)),
            pl.BlockSpec((D, M), lambda i: (0, 0)),
        ],
        out_shape=[
            jax.ShapeDtypeStruct((B, D), jnp.bfloat16),
            jax.ShapeDtypeStruct((D, M), jnp.bfloat16),
        ],
        scratch_shapes=[
            pltpu.VMEM((CP, D), jnp.float32),
            pltpu.VMEM((CP, 1), jnp.float32),
        ],
        compiler_params=pltpu.CompilerParams(
            dimension_semantics=("arbitrary",)),
    )(feat, lbl3, mem_p, src_p)

    loss2d = pl.pallas_call(
        _loss_body,
        grid=(NBLK,),
        in_specs=[
            pl.BlockSpec((RB, D), lambda i: (i, 0)),
            pl.BlockSpec((D, M), lambda i: (0, 0)),
            pl.BlockSpec((1, 1, RB), lambda i: (i, 0, 0)),
        ],
        out_specs=pl.BlockSpec((1, 1), lambda i: (0, 0)),
        out_shape=jax.ShapeDtypeStruct((1, 1), jnp.float32),
        scratch_shapes=[pltpu.VMEM((8, RB // 8), jnp.float32)],
        compiler_params=pltpu.CompilerParams(
            dimension_semantics=("arbitrary",)),
    )(featn, memot, lbl3)

    return loss2d[0, 0]
